# baseline (speedup 1.0000x reference)
import functools

import jax
import jax.numpy as jnp
from jax import lax
from jax.experimental import pallas as pl
from jax.experimental.pallas import tpu as pltpu
from jax.experimental.pallas import tpu_sc as plsc

LANES = 16
UNROLL = 4


def _build_sc_call(B, L, E):
    ROW_IN = L * E
    ROW_OUT = (2 * L + 1) * E
    HALF1 = (L + 1) * E
    info = plsc.get_sparse_core_info()
    NC, NS = info.num_cores, info.num_subcores
    NW = NC * NS
    assert B % (2 * NW) == 0
    PB = B // NW
    nv = E // LANES

    def body(x1_hbm, x2_hbm, cls_hbm, type_hbm, out_hbm,
             in1a, in1b, in2a, in2b, o1a, o1b, o2a, o2b, tbuf,
             sin0, sin1, sout0, sout1):
        wid = lax.axis_index("s") * NC + lax.axis_index("c")
        base_b = wid * PB
        in1 = (in1a, in1b)
        in2 = (in2a, in2b)
        o1 = (o1a, o1b)
        o2 = (o2a, o2b)
        sin = (sin0, sin1)
        sout = (sout0, sout1)

        pltpu.sync_copy(type_hbm, tbuf)
        pltpu.sync_copy(cls_hbm, o1a.at[pl.ds(0, E)])
        pltpu.sync_copy(cls_hbm, o1b.at[pl.ds(0, E)])
        t0 = [tbuf[pl.ds(k * LANES, LANES)] for k in range(nv)]
        t1 = [tbuf[pl.ds(E + k * LANES, LANES)] for k in range(nv)]

        def issue_in(i, b):
            gb = base_b + i
            pltpu.async_copy(x1_hbm.at[pl.ds(gb * ROW_IN, ROW_IN)],
                             in1[b], sin[b])
            pltpu.async_copy(x2_hbm.at[pl.ds(gb * ROW_IN, ROW_IN)],
                             in2[b], sin[b])

        def wait_in(b):
            pltpu.make_async_copy(x1_hbm.at[pl.ds(0, ROW_IN)],
                                  in1[b], sin[b]).wait()
            pltpu.make_async_copy(x2_hbm.at[pl.ds(0, ROW_IN)],
                                  in2[b], sin[b]).wait()

        def issue_out(i, b):
            gb = base_b + i
            pltpu.async_copy(o1[b],
                             out_hbm.at[pl.ds(gb * ROW_OUT, HALF1)], sout[b])
            pltpu.async_copy(o2[b],
                             out_hbm.at[pl.ds(gb * ROW_OUT + HALF1, ROW_IN)],
                             sout[b])

        def wait_out(b):
            pltpu.make_async_copy(o1[b],
                                  out_hbm.at[pl.ds(0, HALF1)], sout[b]).wait()
            pltpu.make_async_copy(o2[b],
                                  out_hbm.at[pl.ds(0, ROW_IN)], sout[b]).wait()

        def compute(b):
            r1, r2 = in1[b], in2[b]
            w1, w2 = o1[b], o2[b]

            @plsc.parallel_loop(0, L, step=1, unroll=UNROLL)
            def _(l):
                base = l * E
                for k in range(nv):
                    off = base + k * LANES
                    s = pl.ds(off, LANES)
                    s1 = pl.ds(E + off, LANES)
                    w1[s1] = r1[s] + t0[k]
                    w2[s] = r2[s] + t1[k]

        issue_in(0, 0)
        issue_in(1, 1)

        def loop_body(g, c):
            for b in range(2):
                i = g * 2 + b
                wait_in(b)

                @pl.when(g > 0)
                def _():
                    wait_out(b)

                compute(b)
                issue_out(i, b)

                @pl.when(g < PB // 2 - 1)
                def _():
                    issue_in(i + 2, b)
            return c

        lax.fori_loop(0, PB // 2, loop_body, 0)
        wait_out(0)
        wait_out(1)

    mesh = plsc.VectorSubcoreMesh(core_axis_name="c", subcore_axis_name="s")
    return pl.kernel(
        body,
        mesh=mesh,
        out_type=jax.ShapeDtypeStruct((B * ROW_OUT,), jnp.float32),
        scratch_types=[
            pltpu.VMEM((ROW_IN,), jnp.float32),
            pltpu.VMEM((ROW_IN,), jnp.float32),
            pltpu.VMEM((ROW_IN,), jnp.float32),
            pltpu.VMEM((ROW_IN,), jnp.float32),
            pltpu.VMEM((HALF1,), jnp.float32),
            pltpu.VMEM((HALF1,), jnp.float32),
            pltpu.VMEM((ROW_IN,), jnp.float32),
            pltpu.VMEM((ROW_IN,), jnp.float32),
            pltpu.VMEM((2 * E,), jnp.float32),
            pltpu.SemaphoreType.DMA,
            pltpu.SemaphoreType.DMA,
            pltpu.SemaphoreType.DMA,
            pltpu.SemaphoreType.DMA,
        ],
    )


def kernel(x1, x2, cls_token, type_embeddings):
    B, L, E = x1.shape
    call = _build_sc_call(B, L, E)
    out_flat = call(
        x1.reshape(-1),
        x2.reshape(-1),
        cls_token.reshape(-1),
        type_embeddings.reshape(-1),
    )
    return out_flat.reshape(B, 2 * L + 1, E)


# speedup vs baseline: 5.6145x; 5.6145x over previous
import functools

import jax
import jax.numpy as jnp
from jax import lax
from jax.experimental import pallas as pl
from jax.experimental.pallas import tpu as pltpu
from jax.experimental.pallas import tpu_sc as plsc

LANES = 16
CHUNK = 16


def _build_sc_call(B, L, E):
    RIN = L * E
    ROUT = (2 * L + 1) * E
    info = plsc.get_sparse_core_info()
    NC, NS = info.num_cores, info.num_subcores
    NW = NC * NS
    NCHUNK = RIN // CHUNK
    assert RIN % CHUNK == 0 and NCHUNK % NW == 0 and E % LANES == 0
    CPW = NCHUNK // NW
    NVC = B // LANES
    assert E <= 2 * CHUNK * LANES

    def body(x1_hbm, x2_hbm, pat_hbm, out_hbm,
             ia, ib, oa, ob, pat,
             sina, sinb, souta, soutb):
        wid = lax.axis_index("s") * NC + lax.axis_index("c")
        ibuf = (ia, ib)
        obuf = (oa, ob)
        sin = (sina, sinb)
        sout = (souta, soutb)
        srcs = (x1_hbm, x2_hbm)
        outoff = (E, E + RIN)
        patbase = (0, E * LANES)

        pltpu.sync_copy(pat_hbm, pat)

        def rs_of(c):
            return (wid * CPW + c) * CHUNK

        def issue_in(c, p):
            pltpu.async_copy(srcs[p].at[pl.ds(rs_of(c), CHUNK)],
                             ibuf[p], sin[p])

        def wait_in(p):
            pltpu.make_async_copy(srcs[p].at[pl.ds(0, CHUNK)],
                                  ibuf[p], sin[p]).wait()

        def issue_out(c, p):
            pltpu.async_copy(obuf[p],
                             out_hbm.at[pl.ds(outoff[p] + rs_of(c), CHUNK)],
                             sout[p])

        def wait_out(p):
            pltpu.make_async_copy(obuf[p],
                                  out_hbm.at[pl.ds(0, CHUNK)],
                                  sout[p]).wait()

        def compute(c, p):
            r, w = ibuf[p], obuf[p]
            pb = patbase[p] + (rs_of(c) & (E - 1)) * LANES
            splats = [pat[pl.ds(pb + j * LANES, LANES)] for j in range(CHUNK)]

            @plsc.parallel_loop(0, NVC, step=1, unroll=2)
            def _(v):
                s = pl.ds(v * LANES, LANES)
                for j in range(CHUNK):
                    w[j, s] = r[j, s] + splats[j]

        issue_in(0, 0)
        issue_in(0, 1)

        def loop_body(c, carry):
            for p in range(2):
                wait_in(p)

                @pl.when(c > 0)
                def _():
                    wait_out(p)

                compute(c, p)
                issue_out(c, p)

                @pl.when(c < CPW - 1)
                def _():
                    issue_in(c + 1, p)
            return carry

        lax.fori_loop(0, CPW, loop_body, 0)
        wait_out(0)
        wait_out(1)

        @pl.when(wid < E // CHUNK)
        def _():
            base = wid * CHUNK

            @plsc.parallel_loop(0, NVC, step=1, unroll=2)
            def _(v):
                s = pl.ds(v * LANES, LANES)
                for j in range(CHUNK):
                    oa[j, s] = pat[pl.ds((2 * E + base + j) * LANES, LANES)]

            pltpu.sync_copy(oa, out_hbm.at[pl.ds(base, CHUNK)])

    mesh = plsc.VectorSubcoreMesh(core_axis_name="c", subcore_axis_name="s")
    return pl.kernel(
        body,
        mesh=mesh,
        out_type=jax.ShapeDtypeStruct((ROUT, B), jnp.float32),
        scratch_types=[
            pltpu.VMEM((CHUNK, B), jnp.float32),
            pltpu.VMEM((CHUNK, B), jnp.float32),
            pltpu.VMEM((CHUNK, B), jnp.float32),
            pltpu.VMEM((CHUNK, B), jnp.float32),
            pltpu.VMEM((3 * E * LANES,), jnp.float32),
            pltpu.SemaphoreType.DMA,
            pltpu.SemaphoreType.DMA,
            pltpu.SemaphoreType.DMA,
            pltpu.SemaphoreType.DMA,
        ],
    )


def kernel(x1, x2, cls_token, type_embeddings):
    B, L, E = x1.shape
    call = _build_sc_call(B, L, E)
    x1v = x1.transpose(1, 2, 0).reshape(L * E, B)
    x2v = x2.transpose(1, 2, 0).reshape(L * E, B)
    scal = jnp.concatenate(
        [type_embeddings.reshape(2 * E), cls_token.reshape(E)])
    pat = jnp.repeat(scal, LANES)
    outv = call(x1v, x2v, pat)
    return outv.reshape(2 * L + 1, E, B).transpose(2, 0, 1)
